# Initial kernel scaffold; baseline (speedup 1.0000x reference)
import jax
import jax.numpy as jnp
from jax.experimental import pallas as pl


def kernel(users_emb, items_emb, adj_indices, adj_values, labels):
    raise NotImplementedError("write your pallas kernel here")


# speedup vs baseline: 6.6377x; 6.6377x over previous
import functools

import jax
import jax.numpy as jnp
from jax import lax
from jax.experimental import pallas as pl
from jax.experimental.pallas import tpu as pltpu
from jax.experimental.pallas import tpu_sc as plsc

U_ROWS = 5000
I_ROWS = 5000
N_ROWS = U_ROWS + I_ROWS
D = 16
NC = 2
NS = 16
CHUNK = 128


def _spmm_body(wide, nchunks, tab_ref, src_ref, dst_ref, val_ref, out_ref,
               src_v, dst_v, val_v, rows_v, msg_v, zeros_v, out_sh, sem):
    cid = lax.axis_index("c")
    sid = lax.axis_index("s")
    rpt = N_ROWS // NS

    def zero_body(i, c):
        zeros_v[i, :] = jnp.zeros((D,), jnp.float32)
        return c
    lax.fori_loop(0, rpt, zero_body, 0)
    pltpu.sync_copy(zeros_v, out_sh.at[pl.ds(sid * rpt, rpt)])

    pltpu.sync_copy(src_ref.at[cid, sid], src_v)
    pltpu.sync_copy(dst_ref.at[cid, sid], dst_v)
    pltpu.sync_copy(val_ref.at[cid, sid], val_v)
    plsc.subcore_barrier()

    def chunk_body(j, c):
        pltpu.async_copy(tab_ref.at[src_v.at[j]], rows_v, sem).wait()

        def scale_group(g, c2):
            base = g * 16
            vblk = val_v[j, pl.ds(base, 16)]
            for k in range(16):
                ee = base + k
                b = jnp.broadcast_to(vblk[k], (16,))
                r = rows_v[ee, pl.ds(0, D)]
                if wide:
                    r = r + rows_v[ee, pl.ds(D, D)]
                msg_v[ee, :] = r * b
            return c2
        lax.fori_loop(0, CHUNK // 16, scale_group, 0)

        pltpu.sync_copy(msg_v, out_sh.at[dst_v.at[j]], add=True)
        return c
    lax.fori_loop(0, nchunks, chunk_body, 0)
    plsc.subcore_barrier()

    pltpu.sync_copy(out_sh.at[pl.ds(sid * rpt, rpt)],
                    out_ref.at[pl.ds(sid * rpt, rpt), cid])


def _make_spmm(wide, nchunks):
    w = 2 * D if wide else D
    mesh = plsc.VectorSubcoreMesh(core_axis_name="c", subcore_axis_name="s",
                                  num_cores=NC, num_subcores=NS)
    return pl.kernel(
        functools.partial(_spmm_body, wide, nchunks),
        out_type=jax.ShapeDtypeStruct((N_ROWS, NC, D), jnp.float32),
        mesh=mesh,
        compiler_params=pltpu.CompilerParams(use_tc_tiling_on_sc=False),
        scratch_types=[
            pltpu.VMEM((nchunks, CHUNK), jnp.int32),
            pltpu.VMEM((nchunks, CHUNK), jnp.int32),
            pltpu.VMEM((nchunks, CHUNK), jnp.float32),
            pltpu.VMEM((CHUNK, w), jnp.float32),
            pltpu.VMEM((CHUNK, D), jnp.float32),
            pltpu.VMEM((N_ROWS // NS, D), jnp.float32),
            pltpu.VMEM_SHARED((N_ROWS, D), jnp.float32),
            pltpu.SemaphoreType.DMA,
        ],
    )


def _mean_body(x0_ref, t1_ref, t2_ref, t3_ref, u_ref, i_ref):
    s = x0_ref[...]
    for t in (t1_ref, t2_ref, t3_ref):
        s = s + t[:, :D] + t[:, D:]
    light = s * 0.25
    u_ref[...] = light[:U_ROWS]
    i_ref[...] = light[U_ROWS:]


def _loss_body(u_ref, it_ref, lab_ref, loss_ref):
    i = pl.program_id(0)
    s = lax.dot_general(u_ref[...], it_ref[...], (((1,), (1,)), ((), ())),
                        preferred_element_type=jnp.float32)
    l = lab_ref[...]
    part = jnp.sum(jnp.maximum(s, 0.0) - s * l
                   + jnp.log1p(jnp.exp(-jnp.abs(s))))

    @pl.when(i == 0)
    def _():
        loss_ref[...] = jnp.zeros_like(loss_ref)

    loss_ref[...] = loss_ref[...] + part

    @pl.when(i == pl.num_programs(0) - 1)
    def _():
        loss_ref[...] = loss_ref[...] * (1.0 / (U_ROWS * I_ROWS))


BU = 200


def kernel(users_emb, items_emb, adj_indices, adj_values, labels):
    e = adj_values.shape[0]
    per_tile = -(-e // (NC * NS * CHUNK)) * CHUNK
    e_pad = per_tile * NC * NS
    nchunks = per_tile // CHUNK

    x0 = jnp.concatenate([users_emb, items_emb], axis=0)
    dst = jnp.pad(adj_indices[0], (0, e_pad - e)).reshape(NC, NS, nchunks, CHUNK)
    src = jnp.pad(adj_indices[1], (0, e_pad - e)).reshape(NC, NS, nchunks, CHUNK)
    val = jnp.pad(adj_values, (0, e_pad - e)).reshape(NC, NS, nchunks, CHUNK)

    spmm1 = _make_spmm(False, nchunks)
    spmm_w = _make_spmm(True, nchunks)
    t1 = spmm1(x0, src, dst, val).reshape(N_ROWS, NC * D)
    t2 = spmm_w(t1, src, dst, val).reshape(N_ROWS, NC * D)
    t3 = spmm_w(t2, src, dst, val).reshape(N_ROWS, NC * D)

    user_all, item_all = pl.pallas_call(
        _mean_body,
        out_shape=[jax.ShapeDtypeStruct((U_ROWS, D), jnp.float32),
                   jax.ShapeDtypeStruct((I_ROWS, D), jnp.float32)],
    )(x0, t1, t2, t3)

    loss = pl.pallas_call(
        _loss_body,
        grid=(U_ROWS // BU,),
        in_specs=[
            pl.BlockSpec((BU, D), lambda i: (i, 0)),
            pl.BlockSpec((I_ROWS, D), lambda i: (0, 0)),
            pl.BlockSpec((BU, I_ROWS), lambda i: (i, 0)),
        ],
        out_specs=pl.BlockSpec((1, 1), lambda i: (0, 0)),
        out_shape=jax.ShapeDtypeStruct((1, 1), jnp.float32),
    )(user_all, item_all, labels)

    return (loss[0, 0], user_all, item_all)


# speedup vs baseline: 13.4135x; 2.0208x over previous
import functools

import jax
import jax.numpy as jnp
from jax import lax
from jax.experimental import pallas as pl
from jax.experimental.pallas import tpu as pltpu
from jax.experimental.pallas import tpu_sc as plsc

U_ROWS = 5000
I_ROWS = 5000
N_ROWS = U_ROWS + I_ROWS
D = 16
NC = 2
NS = 16
CHUNK = 128
NBUF = 4


def _spmm_body(ntab, nchunks, *refs):
    tabs = refs[:ntab]
    src_ref, dst_ref, val_ref, out0_ref, out1_ref = refs[ntab:ntab + 5]
    (src_v, dst_v, val_v, rows0_v, rows1_v, msg_v, zeros_v, out_sh,
     gsem0, gsem1, ssem) = refs[ntab + 5:]
    cid = lax.axis_index("c")
    sid = lax.axis_index("s")
    rpt = N_ROWS // NS

    def zero_body(i, c):
        zeros_v[i, :] = jnp.zeros((D,), jnp.float32)
        return c
    lax.fori_loop(0, rpt, zero_body, 0)
    pltpu.sync_copy(zeros_v, out_sh.at[pl.ds(sid * rpt, rpt)])

    pltpu.sync_copy(src_ref.at[cid, sid], src_v)
    pltpu.sync_copy(dst_ref.at[cid, sid], dst_v)
    pltpu.sync_copy(val_ref.at[cid, sid], val_v)
    plsc.subcore_barrier()

    def fire_gathers(j, b):
        pltpu.async_copy(tabs[0].at[src_v.at[j]], rows0_v.at[b], gsem0.at[b])
        if ntab == 2:
            pltpu.async_copy(tabs[1].at[src_v.at[j]], rows1_v.at[b],
                             gsem1.at[b])

    def wait_gathers(j, b):
        pltpu.make_async_copy(tabs[0].at[src_v.at[j]], rows0_v.at[b],
                              gsem0.at[b]).wait()
        if ntab == 2:
            pltpu.make_async_copy(tabs[1].at[src_v.at[j]], rows1_v.at[b],
                                  gsem1.at[b]).wait()

    for b in range(NBUF):
        fire_gathers(b, b)

    def super_body(jj, c):
        for b in range(NBUF):
            j = jj * NBUF + b
            wait_gathers(j, b)

            @pl.when(jj > 0)
            def _():
                pltpu.make_async_copy(msg_v.at[b], out_sh.at[dst_v.at[j]],
                                      ssem.at[b]).wait()

            def scale_group(g, c2):
                base = g * 16
                vblk = val_v[j, pl.ds(base, 16)]
                for k in range(16):
                    ee = base + k
                    vb = jnp.broadcast_to(vblk[k], (16,))
                    r = rows0_v[b, ee, :]
                    if ntab == 2:
                        r = r + rows1_v[b, ee, :]
                    msg_v[b, ee, :] = r * vb
                return c2
            lax.fori_loop(0, CHUNK // 16, scale_group, 0)

            @pl.when(j + NBUF < nchunks)
            def _():
                fire_gathers(j + NBUF, b)

            pltpu.async_copy(msg_v.at[b], out_sh.at[dst_v.at[j]], ssem.at[b],
                             add=True)
        return c
    lax.fori_loop(0, nchunks // NBUF, super_body, 0)

    for b in range(NBUF):
        pltpu.make_async_copy(msg_v.at[b], out_sh.at[dst_v.at[b]],
                              ssem.at[b]).wait()
    plsc.subcore_barrier()

    @pl.when(cid == 0)
    def _():
        pltpu.sync_copy(out_sh.at[pl.ds(sid * rpt, rpt)],
                        out0_ref.at[pl.ds(sid * rpt, rpt)])

    @pl.when(cid == 1)
    def _():
        pltpu.sync_copy(out_sh.at[pl.ds(sid * rpt, rpt)],
                        out1_ref.at[pl.ds(sid * rpt, rpt)])


def _make_spmm(ntab, nchunks):
    mesh = plsc.VectorSubcoreMesh(core_axis_name="c", subcore_axis_name="s",
                                  num_cores=NC, num_subcores=NS)
    part = jax.ShapeDtypeStruct((N_ROWS, D), jnp.float32)
    return pl.kernel(
        functools.partial(_spmm_body, ntab, nchunks),
        out_type=[part, part],
        mesh=mesh,
        compiler_params=pltpu.CompilerParams(use_tc_tiling_on_sc=False),
        scratch_types=[
            pltpu.VMEM((nchunks, CHUNK), jnp.int32),
            pltpu.VMEM((nchunks, CHUNK), jnp.int32),
            pltpu.VMEM((nchunks, CHUNK), jnp.float32),
            pltpu.VMEM((NBUF, CHUNK, D), jnp.float32),
            pltpu.VMEM((NBUF, CHUNK, D), jnp.float32),
            pltpu.VMEM((NBUF, CHUNK, D), jnp.float32),
            pltpu.VMEM((N_ROWS // NS, D), jnp.float32),
            pltpu.VMEM_SHARED((N_ROWS, D), jnp.float32),
            pltpu.SemaphoreType.DMA((NBUF,)),
            pltpu.SemaphoreType.DMA((NBUF,)),
            pltpu.SemaphoreType.DMA((NBUF,)),
        ],
    )


def _mean_body(*refs):
    parts = refs[:7]
    u_ref, i_ref = refs[7:]
    s = parts[0][...]
    for p in parts[1:]:
        s = s + p[...]
    light = s * 0.25
    u_ref[...] = light[:U_ROWS]
    i_ref[...] = light[U_ROWS:]


LOG2E = 1.4426950408889634
LN2 = 0.6931471805599453


def _loss_body(u_ref, it_ref, lab_ref, loss_ref):
    i = pl.program_id(0)
    u = u_ref[...]
    it = it_ref[...]
    s = lax.dot_general(u, it, (((1,), (1,)), ((), ())),
                        preferred_element_type=jnp.float32)
    t = jnp.exp2(jnp.abs(s) * (-LOG2E))
    soft = jnp.maximum(s, 0.0) + jnp.log2(1.0 + t) * LN2
    part = jnp.sum(soft - s * lab_ref[...])

    @pl.when(i == 0)
    def _():
        loss_ref[...] = jnp.zeros_like(loss_ref)

    loss_ref[...] = loss_ref[...] + part

    @pl.when(i == pl.num_programs(0) - 1)
    def _():
        loss_ref[...] = loss_ref[...] * (1.0 / (U_ROWS * I_ROWS))


BU = 200


def kernel(users_emb, items_emb, adj_indices, adj_values, labels):
    e = adj_values.shape[0]
    ring = NBUF * CHUNK
    per_tile = -(-e // (NC * NS * ring)) * ring
    e_pad = per_tile * NC * NS
    nchunks = per_tile // CHUNK

    x0 = jnp.concatenate([users_emb, items_emb], axis=0)
    dst = jnp.pad(adj_indices[0], (0, e_pad - e)).reshape(NC, NS, nchunks, CHUNK)
    src = jnp.pad(adj_indices[1], (0, e_pad - e)).reshape(NC, NS, nchunks, CHUNK)
    val = jnp.pad(adj_values, (0, e_pad - e)).reshape(NC, NS, nchunks, CHUNK)

    spmm1 = _make_spmm(1, nchunks)
    spmm2 = _make_spmm(2, nchunks)
    p10, p11 = spmm1(x0, src, dst, val)
    p20, p21 = spmm2(p10, p11, src, dst, val)
    p30, p31 = spmm2(p20, p21, src, dst, val)

    user_all, item_all = pl.pallas_call(
        _mean_body,
        out_shape=[jax.ShapeDtypeStruct((U_ROWS, D), jnp.float32),
                   jax.ShapeDtypeStruct((I_ROWS, D), jnp.float32)],
    )(x0, p10, p11, p20, p21, p30, p31)

    loss = pl.pallas_call(
        _loss_body,
        grid=(U_ROWS // BU,),
        in_specs=[
            pl.BlockSpec((BU, D), lambda i: (i, 0)),
            pl.BlockSpec((I_ROWS, D), lambda i: (0, 0)),
            pl.BlockSpec((BU, I_ROWS), lambda i: (i, 0)),
        ],
        out_specs=pl.BlockSpec((1, 1), lambda i: (0, 0)),
        out_shape=jax.ShapeDtypeStruct((1, 1), jnp.float32),
    )(user_all, item_all, labels)

    return (loss[0, 0], user_all, item_all)


# speedup vs baseline: 13.6288x; 1.0161x over previous
import functools

import jax
import jax.numpy as jnp
from jax import lax
from jax.experimental import pallas as pl
from jax.experimental.pallas import tpu as pltpu
from jax.experimental.pallas import tpu_sc as plsc

U_ROWS = 5000
I_ROWS = 5000
N_ROWS = U_ROWS + I_ROWS
D = 16
NC = 2
NS = 16
CHUNK = 128
NBUF = 4


_GATHER_DN = lax.GatherDimensionNumbers(
    offset_dims=(), collapsed_slice_dims=(0,), start_index_map=(0,))


def _lane_bcast(v, k):
    idx = jnp.full((16, 1), k, jnp.int32)
    return lax.gather(v, idx, _GATHER_DN, (1,),
                      mode=lax.GatherScatterMode.PROMISE_IN_BOUNDS)


def _spmm_body(ntab, nchunks, *refs):
    tabs = refs[:ntab]
    src_ref, dst_ref, val_ref, out0_ref, out1_ref = refs[ntab:ntab + 5]
    (src_v, dst_v, val_v, rows0_v, rows1_v, msg_v, zeros_v, out_sh,
     gsem0, gsem1, ssem) = refs[ntab + 5:]
    cid = lax.axis_index("c")
    sid = lax.axis_index("s")
    rpt = N_ROWS // NS

    def zero_body(i, c):
        zeros_v[i, :] = jnp.zeros((D,), jnp.float32)
        return c
    lax.fori_loop(0, rpt, zero_body, 0)
    pltpu.sync_copy(zeros_v, out_sh.at[pl.ds(sid * rpt, rpt)])

    pltpu.sync_copy(src_ref.at[cid, sid], src_v)
    pltpu.sync_copy(dst_ref.at[cid, sid], dst_v)
    pltpu.sync_copy(val_ref.at[cid, sid], val_v)
    plsc.subcore_barrier()

    def fire_gathers(j, b):
        pltpu.async_copy(tabs[0].at[src_v.at[j]], rows0_v.at[b], gsem0.at[b])
        if ntab == 2:
            pltpu.async_copy(tabs[1].at[src_v.at[j]], rows1_v.at[b],
                             gsem1.at[b])

    def wait_gathers(j, b):
        pltpu.make_async_copy(tabs[0].at[src_v.at[j]], rows0_v.at[b],
                              gsem0.at[b]).wait()
        if ntab == 2:
            pltpu.make_async_copy(tabs[1].at[src_v.at[j]], rows1_v.at[b],
                                  gsem1.at[b]).wait()

    for b in range(NBUF):
        fire_gathers(b, b)

    def super_body(jj, c):
        for b in range(NBUF):
            j = jj * NBUF + b
            wait_gathers(j, b)

            @pl.when(jj > 0)
            def _():
                pltpu.make_async_copy(msg_v.at[b], out_sh.at[dst_v.at[j]],
                                      ssem.at[b]).wait()

            def scale_group(g, c2):
                base = g * 16
                vblk = val_v[j, pl.ds(base, 16)]
                for k in range(16):
                    ee = base + k
                    vb = _lane_bcast(vblk, k)
                    r = rows0_v[b, ee, :]
                    if ntab == 2:
                        r = r + rows1_v[b, ee, :]
                    msg_v[b, ee, :] = r * vb
                return c2
            lax.fori_loop(0, CHUNK // 16, scale_group, 0)

            @pl.when(j + NBUF < nchunks)
            def _():
                fire_gathers(j + NBUF, b)

            pltpu.async_copy(msg_v.at[b], out_sh.at[dst_v.at[j]], ssem.at[b],
                             add=True)
        return c
    lax.fori_loop(0, nchunks // NBUF, super_body, 0)

    for b in range(NBUF):
        pltpu.make_async_copy(msg_v.at[b], out_sh.at[dst_v.at[b]],
                              ssem.at[b]).wait()
    plsc.subcore_barrier()

    @pl.when(cid == 0)
    def _():
        pltpu.sync_copy(out_sh.at[pl.ds(sid * rpt, rpt)],
                        out0_ref.at[pl.ds(sid * rpt, rpt)])

    @pl.when(cid == 1)
    def _():
        pltpu.sync_copy(out_sh.at[pl.ds(sid * rpt, rpt)],
                        out1_ref.at[pl.ds(sid * rpt, rpt)])


def _make_spmm(ntab, nchunks):
    mesh = plsc.VectorSubcoreMesh(core_axis_name="c", subcore_axis_name="s",
                                  num_cores=NC, num_subcores=NS)
    part = jax.ShapeDtypeStruct((N_ROWS, D), jnp.float32)
    return pl.kernel(
        functools.partial(_spmm_body, ntab, nchunks),
        out_type=[part, part],
        mesh=mesh,
        compiler_params=pltpu.CompilerParams(use_tc_tiling_on_sc=False),
        scratch_types=[
            pltpu.VMEM((nchunks, CHUNK), jnp.int32),
            pltpu.VMEM((nchunks, CHUNK), jnp.int32),
            pltpu.VMEM((nchunks, CHUNK), jnp.float32),
            pltpu.VMEM((NBUF, CHUNK, D), jnp.float32),
            pltpu.VMEM((NBUF, CHUNK, D), jnp.float32),
            pltpu.VMEM((NBUF, CHUNK, D), jnp.float32),
            pltpu.VMEM((N_ROWS // NS, D), jnp.float32),
            pltpu.VMEM_SHARED((N_ROWS, D), jnp.float32),
            pltpu.SemaphoreType.DMA((NBUF,)),
            pltpu.SemaphoreType.DMA((NBUF,)),
            pltpu.SemaphoreType.DMA((NBUF,)),
        ],
    )


def _mean_body(*refs):
    parts = refs[:7]
    u_ref, i_ref = refs[7:]
    s = parts[0][...]
    for p in parts[1:]:
        s = s + p[...]
    light = s * 0.25
    u_ref[...] = light[:U_ROWS]
    i_ref[...] = light[U_ROWS:]


LOG2E = 1.4426950408889634
LN2 = 0.6931471805599453


def _loss_body(u_ref, it_ref, lab_ref, loss_ref):
    i = pl.program_id(0)
    u = u_ref[...]
    it = it_ref[...]
    s = lax.dot_general(u, it, (((1,), (1,)), ((), ())),
                        preferred_element_type=jnp.float32)
    t = jnp.exp2(jnp.abs(s) * (-LOG2E))
    soft = jnp.maximum(s, 0.0) + jnp.log2(1.0 + t) * LN2
    part = jnp.sum(soft - s * lab_ref[...])

    @pl.when(i == 0)
    def _():
        loss_ref[...] = jnp.zeros_like(loss_ref)

    loss_ref[...] = loss_ref[...] + part

    @pl.when(i == pl.num_programs(0) - 1)
    def _():
        loss_ref[...] = loss_ref[...] * (1.0 / (U_ROWS * I_ROWS))


BU = 200


def kernel(users_emb, items_emb, adj_indices, adj_values, labels):
    e = adj_values.shape[0]
    ring = NBUF * CHUNK
    per_tile = -(-e // (NC * NS * ring)) * ring
    e_pad = per_tile * NC * NS
    nchunks = per_tile // CHUNK

    x0 = jnp.concatenate([users_emb, items_emb], axis=0)
    dst = jnp.pad(adj_indices[0], (0, e_pad - e)).reshape(NC, NS, nchunks, CHUNK)
    src = jnp.pad(adj_indices[1], (0, e_pad - e)).reshape(NC, NS, nchunks, CHUNK)
    val = jnp.pad(adj_values, (0, e_pad - e)).reshape(NC, NS, nchunks, CHUNK)

    spmm1 = _make_spmm(1, nchunks)
    spmm2 = _make_spmm(2, nchunks)
    p10, p11 = spmm1(x0, src, dst, val)
    p20, p21 = spmm2(p10, p11, src, dst, val)
    p30, p31 = spmm2(p20, p21, src, dst, val)

    user_all, item_all = pl.pallas_call(
        _mean_body,
        out_shape=[jax.ShapeDtypeStruct((U_ROWS, D), jnp.float32),
                   jax.ShapeDtypeStruct((I_ROWS, D), jnp.float32)],
    )(x0, p10, p11, p20, p21, p30, p31)

    loss = pl.pallas_call(
        _loss_body,
        grid=(U_ROWS // BU,),
        in_specs=[
            pl.BlockSpec((BU, D), lambda i: (i, 0)),
            pl.BlockSpec((I_ROWS, D), lambda i: (0, 0)),
            pl.BlockSpec((BU, I_ROWS), lambda i: (i, 0)),
        ],
        out_specs=pl.BlockSpec((1, 1), lambda i: (0, 0)),
        out_shape=jax.ShapeDtypeStruct((1, 1), jnp.float32),
    )(user_all, item_all, labels)

    return (loss[0, 0], user_all, item_all)


# speedup vs baseline: 14.0095x; 1.0279x over previous
import functools

import jax
import jax.numpy as jnp
from jax import lax
from jax.experimental import pallas as pl
from jax.experimental.pallas import tpu as pltpu
from jax.experimental.pallas import tpu_sc as plsc

U_ROWS = 5000
I_ROWS = 5000
N_ROWS = U_ROWS + I_ROWS
D = 16
NC = 2
NS = 16
CHUNK = 128
NBUF = 4
NCH0 = 108
NCH1 = 52


_GATHER_DN = lax.GatherDimensionNumbers(
    offset_dims=(), collapsed_slice_dims=(0,), start_index_map=(0,))


def _lane_bcast(v, k):
    idx = jnp.full((16, 1), k, jnp.int32)
    return lax.gather(v, idx, _GATHER_DN, (1,),
                      mode=lax.GatherScatterMode.PROMISE_IN_BOUNDS)


def _spmm_body(ntab, *refs):
    tabs = refs[:ntab]
    src_ref, dst_ref, val_ref, out0_ref, out1_ref = refs[ntab:ntab + 5]
    (src_v, dst_v, val_v, rows0_v, rows1_v, msg_v, zeros_v, out_sh,
     gsem0, gsem1, ssem) = refs[ntab + 5:]
    cid = lax.axis_index("c")
    sid = lax.axis_index("s")
    rpt = N_ROWS // NS

    def zero_body(i, c):
        zeros_v[i, :] = jnp.zeros((D,), jnp.float32)
        return c
    lax.fori_loop(0, rpt, zero_body, 0)
    pltpu.sync_copy(zeros_v, out_sh.at[pl.ds(sid * rpt, rpt)])

    nch = jnp.where(cid == 0, NCH0, NCH1)

    @pl.when(cid == 0)
    def _():
        base = sid * NCH0
        pltpu.sync_copy(src_ref.at[pl.ds(base, NCH0)], src_v)
        pltpu.sync_copy(dst_ref.at[pl.ds(base, NCH0)], dst_v)
        pltpu.sync_copy(val_ref.at[pl.ds(base, NCH0)], val_v)

    @pl.when(cid == 1)
    def _():
        base = NS * NCH0 + sid * NCH1
        pltpu.sync_copy(src_ref.at[pl.ds(base, NCH1)],
                        src_v.at[pl.ds(0, NCH1)])
        pltpu.sync_copy(dst_ref.at[pl.ds(base, NCH1)],
                        dst_v.at[pl.ds(0, NCH1)])
        pltpu.sync_copy(val_ref.at[pl.ds(base, NCH1)],
                        val_v.at[pl.ds(0, NCH1)])

    plsc.subcore_barrier()

    def fire_gathers(j, b):
        pltpu.async_copy(tabs[0].at[src_v.at[j]], rows0_v.at[b], gsem0.at[b])
        if ntab == 2:
            pltpu.async_copy(tabs[1].at[src_v.at[j]], rows1_v.at[b],
                             gsem1.at[b])

    def wait_gathers(j, b):
        pltpu.make_async_copy(tabs[0].at[src_v.at[j]], rows0_v.at[b],
                              gsem0.at[b]).wait()
        if ntab == 2:
            pltpu.make_async_copy(tabs[1].at[src_v.at[j]], rows1_v.at[b],
                                  gsem1.at[b]).wait()

    for b in range(NBUF):
        fire_gathers(b, b)

    def super_body(jj, c):
        for b in range(NBUF):
            j = jj * NBUF + b
            wait_gathers(j, b)

            @pl.when(jj > 0)
            def _():
                pltpu.make_async_copy(msg_v.at[b], out_sh.at[dst_v.at[j]],
                                      ssem.at[b]).wait()

            def scale_group(g, c2):
                base = g * 16
                vblk = val_v[j, pl.ds(base, 16)]
                for k in range(16):
                    ee = base + k
                    vb = _lane_bcast(vblk, k)
                    r = rows0_v[b, ee, :]
                    if ntab == 2:
                        r = r + rows1_v[b, ee, :]
                    msg_v[b, ee, :] = r * vb
                return c2
            lax.fori_loop(0, CHUNK // 16, scale_group, 0)

            @pl.when(j + NBUF < nch)
            def _():
                fire_gathers(j + NBUF, b)

            pltpu.async_copy(msg_v.at[b], out_sh.at[dst_v.at[j]], ssem.at[b],
                             add=True)
        return c
    lax.fori_loop(0, nch // NBUF, super_body, 0)

    for b in range(NBUF):
        pltpu.make_async_copy(msg_v.at[b], out_sh.at[dst_v.at[b]],
                              ssem.at[b]).wait()
    plsc.subcore_barrier()

    @pl.when(cid == 0)
    def _():
        pltpu.sync_copy(out_sh.at[pl.ds(sid * rpt, rpt)],
                        out0_ref.at[pl.ds(sid * rpt, rpt)])

    @pl.when(cid == 1)
    def _():
        pltpu.sync_copy(out_sh.at[pl.ds(sid * rpt, rpt)],
                        out1_ref.at[pl.ds(sid * rpt, rpt)])


def _make_spmm(ntab):
    mesh = plsc.VectorSubcoreMesh(core_axis_name="c", subcore_axis_name="s",
                                  num_cores=NC, num_subcores=NS)
    part = jax.ShapeDtypeStruct((N_ROWS, D), jnp.float32)
    return pl.kernel(
        functools.partial(_spmm_body, ntab),
        out_type=[part, part],
        mesh=mesh,
        compiler_params=pltpu.CompilerParams(use_tc_tiling_on_sc=False),
        scratch_types=[
            pltpu.VMEM((NCH0, CHUNK), jnp.int32),
            pltpu.VMEM((NCH0, CHUNK), jnp.int32),
            pltpu.VMEM((NCH0, CHUNK), jnp.float32),
            pltpu.VMEM((NBUF, CHUNK, D), jnp.float32),
            pltpu.VMEM((NBUF, CHUNK, D), jnp.float32),
            pltpu.VMEM((NBUF, CHUNK, D), jnp.float32),
            pltpu.VMEM((N_ROWS // NS, D), jnp.float32),
            pltpu.VMEM_SHARED((N_ROWS, D), jnp.float32),
            pltpu.SemaphoreType.DMA((NBUF,)),
            pltpu.SemaphoreType.DMA((NBUF,)),
            pltpu.SemaphoreType.DMA((NBUF,)),
        ],
    )


def _mean_body(*refs):
    parts = refs[:7]
    u_ref, i_ref = refs[7:]
    s = parts[0][...]
    for p in parts[1:]:
        s = s + p[...]
    light = s * 0.25
    u_ref[...] = light[:U_ROWS]
    i_ref[...] = light[U_ROWS:]


LOG2E = 1.4426950408889634
LN2 = 0.6931471805599453


def _loss_body(u_ref, it_ref, lab_ref, loss_ref):
    i = pl.program_id(0)
    u = u_ref[...]
    it = it_ref[...]
    s2 = lax.dot_general(u * LOG2E, it, (((1,), (1,)), ((), ())),
                         preferred_element_type=jnp.float32)
    t = jnp.exp2(-jnp.abs(s2))
    soft = jnp.maximum(s2, 0.0) + jnp.log2(1.0 + t)
    part = LN2 * jnp.sum(soft - s2 * lab_ref[...])

    @pl.when(i == 0)
    def _():
        loss_ref[...] = jnp.zeros_like(loss_ref)

    loss_ref[...] = loss_ref[...] + part

    @pl.when(i == pl.num_programs(0) - 1)
    def _():
        loss_ref[...] = loss_ref[...] * (1.0 / (U_ROWS * I_ROWS))


BU = 200


def kernel(users_emb, items_emb, adj_indices, adj_values, labels):
    e = adj_values.shape[0]
    total_chunks = NS * (NCH0 + NCH1)
    e_pad = total_chunks * CHUNK
    assert e <= e_pad

    x0 = jnp.concatenate([users_emb, items_emb], axis=0)
    dst = jnp.pad(adj_indices[0], (0, e_pad - e)).reshape(total_chunks, CHUNK)
    src = jnp.pad(adj_indices[1], (0, e_pad - e)).reshape(total_chunks, CHUNK)
    val = jnp.pad(adj_values, (0, e_pad - e)).reshape(total_chunks, CHUNK)

    spmm1 = _make_spmm(1)
    spmm2 = _make_spmm(2)
    p10, p11 = spmm1(x0, src, dst, val)
    p20, p21 = spmm2(p10, p11, src, dst, val)
    p30, p31 = spmm2(p20, p21, src, dst, val)

    user_all, item_all = pl.pallas_call(
        _mean_body,
        out_shape=[jax.ShapeDtypeStruct((U_ROWS, D), jnp.float32),
                   jax.ShapeDtypeStruct((I_ROWS, D), jnp.float32)],
    )(x0, p10, p11, p20, p21, p30, p31)

    loss = pl.pallas_call(
        _loss_body,
        grid=(U_ROWS // BU,),
        in_specs=[
            pl.BlockSpec((BU, D), lambda i: (i, 0)),
            pl.BlockSpec((I_ROWS, D), lambda i: (0, 0)),
            pl.BlockSpec((BU, I_ROWS), lambda i: (i, 0)),
        ],
        out_specs=pl.BlockSpec((1, 1), lambda i: (0, 0)),
        out_shape=jax.ShapeDtypeStruct((1, 1), jnp.float32),
    )(user_all, item_all, labels)

    return (loss[0, 0], user_all, item_all)
